# baseline (speedup 1.0000x reference)
import functools

import jax
import jax.numpy as jnp
import numpy as np
from jax import lax
from jax.experimental import pallas as pl
from jax.experimental.pallas import tpu as pltpu
from jax.experimental.pallas import tpu_sc as plsc

N = 10000
E = 320000
D = 128
NC = 2
NS = 16
NW = NC * NS
EPT = E // NW
CH = 128
NPE = 112
NCH = (EPT + NPE) // CH
NACC = N + 16
DUMP = N + 8
NBUF = 5
NG = NCH // NBUF
RPT = N // NS
DGB = 1000


def _sc_aggregate(x, row_idx, col_idx):
    mesh = plsc.VectorSubcoreMesh(
        core_axis_name="c", subcore_axis_name="s", num_cores=NC,
        num_subcores=NS)

    @functools.partial(
        pl.kernel,
        out_type=(
            jax.ShapeDtypeStruct((NC, N, D), jnp.bfloat16),
            jax.ShapeDtypeStruct((NC, N), jnp.float32),
        ),
        mesh=mesh,
        compiler_params=pltpu.CompilerParams(use_tc_tiling_on_sc=False),
        scratch_types=[
            pltpu.VMEM((NCH, CH), jnp.int32),
            pltpu.VMEM((NCH, CH), jnp.int32),
            [pltpu.VMEM((CH, D), jnp.bfloat16)] * NBUF,
            pltpu.VMEM((DGB,), jnp.float32),
            pltpu.VMEM((CH,), jnp.float32),
            pltpu.VMEM_SHARED((NACC, D), jnp.bfloat16),
            pltpu.VMEM_SHARED((NACC,), jnp.float32),
            [pltpu.SemaphoreType.DMA] * NBUF,
        ],
    )
    def k(x_hbm, row_hbm, col_hbm, agg_hbm, deg_hbm, rowi_v, coli_v,
          bufs, zd_v, ones_v, acc_sp, deg_sp, gsem):
        c = lax.axis_index("c")
        s = lax.axis_index("s")
        wid = s * NC + c

        zero16 = jnp.zeros((16,), jnp.float32)
        zero32 = jnp.zeros((32,), jnp.bfloat16)

        def zrow(r, _):
            def zcol(q, _):
                bufs[0][r, pl.ds(q * 32, 32)] = zero32
                return _
            return lax.fori_loop(0, D // 32, zcol, _)
        lax.fori_loop(0, CH, zrow, None)

        def zdeg(r, _):
            zd_v[pl.ds(r * 16, 16)] = zero16
            return _
        lax.fori_loop(0, DGB // 16, zdeg, None)

        def fones(r, _):
            ones_v[pl.ds(r * 16, 16)] = jnp.ones((16,), jnp.float32)
            return _
        lax.fori_loop(0, CH // 16, fones, None)

        def zacc(b, _):
            pltpu.sync_copy(bufs[0],
                            acc_sp.at[pl.ds(s * RPT + b * CH, CH)])
            return _
        lax.fori_loop(0, RPT // CH, zacc, None)
        pltpu.sync_copy(bufs[0].at[pl.ds(0, RPT % CH)],
                        acc_sp.at[pl.ds(s * RPT + RPT - RPT % CH,
                                        RPT % CH)])

        @pl.when(s < N // DGB)
        def _():
            pltpu.sync_copy(zd_v, deg_sp.at[pl.ds(s * DGB, DGB)])

        pltpu.sync_copy(row_hbm.at[wid], rowi_v)
        pltpu.sync_copy(col_hbm.at[wid], coli_v)

        plsc.subcore_barrier()

        def gather(i, j):
            pltpu.async_copy(x_hbm.at[rowi_v.at[i]], bufs[j], gsem[j])

        def wait_gather(i, j):
            pltpu.make_async_copy(x_hbm.at[rowi_v.at[i]], bufs[j],
                                  gsem[j]).wait()

        def scat(i, j):
            pltpu.sync_copy(bufs[j], acc_sp.at[coli_v.at[i]], add=True)
            pltpu.sync_copy(ones_v, deg_sp.at[coli_v.at[i]], add=True)

        for j in range(NBUF):
            gather(j, j)

        def group(gi, _):
            base = gi * NBUF
            for j in range(NBUF):
                wait_gather(base + j, j)
                scat(base + j, j)
                nxt = base + NBUF + j

                @pl.when(nxt < NCH)
                def _(j=j, nxt=nxt):
                    gather(nxt, j)
            return _
        lax.fori_loop(0, NG, group, None)

        for j in range(NCH - NG * NBUF):
            wait_gather(NG * NBUF + j, j)
            scat(NG * NBUF + j, j)

        plsc.subcore_barrier()

        pltpu.sync_copy(acc_sp.at[pl.ds(s * RPT, RPT)],
                        agg_hbm.at[c, pl.ds(s * RPT, RPT)])

        @pl.when(s < N // DGB)
        def _():
            pltpu.sync_copy(deg_sp.at[pl.ds(s * DGB, DGB)],
                            deg_hbm.at[c, pl.ds(s * DGB, DGB)])

    return k(x, row_idx, col_idx)


def _tc_combine_kernel(aggp_ref, degp_ref, x_ref, wn_ref, wr_ref, o_ref):
    inv_sqrt_d = np.float32(1.0 / np.sqrt(D))
    join_scale = np.float32(0.5 / np.sqrt(0.5))

    def norm_w(w):
        nrm = jnp.sqrt(jnp.sum(w * w, axis=1, keepdims=True))
        sc = np.float32(1e-4) + nrm * inv_sqrt_d
        return (w / sc) * inv_sqrt_d

    feat = (aggp_ref[0].astype(jnp.float32)
            + aggp_ref[1].astype(jnp.float32))
    deg = degp_ref[0] + degp_ref[1]
    dinv = jnp.where(deg > 0, lax.rsqrt(jnp.maximum(deg, 1.0)), 0.0)
    agg = feat * dinv

    wn = norm_w(wn_ref[...])
    wr = norm_w(wr_ref[...])
    dn = (((1,), (1,)), ((), ()))
    neigh = lax.dot_general(agg, wn, dn, preferred_element_type=jnp.float32)
    res = lax.dot_general(x_ref[...], wr, dn,
                          preferred_element_type=jnp.float32)
    o_ref[...] = jnp.where(deg > 0, (res + neigh) * join_scale, res)


def _tc_combine(aggp, degp, x, w_neigh, w_res):
    blk = 1000
    grid = N // blk
    return pl.pallas_call(
        _tc_combine_kernel,
        grid=(grid,),
        in_specs=[
            pl.BlockSpec((NC, blk, D), lambda i: (0, i, 0)),
            pl.BlockSpec((NC, blk, 1), lambda i: (0, i, 0)),
            pl.BlockSpec((blk, D), lambda i: (i, 0)),
            pl.BlockSpec((D, D), lambda i: (0, 0)),
            pl.BlockSpec((D, D), lambda i: (0, 0)),
        ],
        out_specs=pl.BlockSpec((blk, D), lambda i: (i, 0)),
        out_shape=jax.ShapeDtypeStruct((N, D), jnp.float32),
    )(aggp, degp, x, w_neigh, w_res)


@jax.jit
def kernel(x, edge_index, W_neigh, W_res):
    x = x.astype(jnp.float32)
    ei = edge_index.astype(jnp.int32)
    row = jnp.concatenate(
        [ei[0].reshape(NW, EPT), jnp.zeros((NW, NPE), jnp.int32)],
        axis=1).reshape(NW, NCH, CH)
    col = jnp.concatenate(
        [ei[1].reshape(NW, EPT), jnp.full((NW, NPE), DUMP, jnp.int32)],
        axis=1).reshape(NW, NCH, CH)
    aggp, degp = _sc_aggregate(x.astype(jnp.bfloat16), row, col)
    return _tc_combine(aggp, degp.reshape(NC, N, 1), x, W_neigh, W_res)


# speedup vs baseline: 1.4676x; 1.4676x over previous
import functools

import jax
import jax.numpy as jnp
import numpy as np
from jax import lax
from jax.experimental import pallas as pl
from jax.experimental.pallas import tpu as pltpu
from jax.experimental.pallas import tpu_sc as plsc

N = 10000
E = 320000
D = 128
NC = 2
NS = 16
NW = NC * NS
EPT = E // NW
CH = 80
NCH = EPT // CH
NBUF = 5
NG = NCH // NBUF
RPT = N // NS
DGB = 1000


def _sc_aggregate(x, row_idx, col_idx):
    mesh = plsc.VectorSubcoreMesh(
        core_axis_name="c", subcore_axis_name="s", num_cores=NC,
        num_subcores=NS)

    @functools.partial(
        pl.kernel,
        out_type=(
            jax.ShapeDtypeStruct((NC, N, D), jnp.bfloat16),
            jax.ShapeDtypeStruct((NC, N), jnp.float32),
        ),
        mesh=mesh,
        compiler_params=pltpu.CompilerParams(use_tc_tiling_on_sc=False),
        scratch_types=[
            pltpu.VMEM((NCH, CH), jnp.int32),
            pltpu.VMEM((NCH, CH), jnp.int32),
            [pltpu.VMEM((CH, D), jnp.bfloat16)] * NBUF,
            pltpu.VMEM((DGB,), jnp.float32),
            pltpu.VMEM((CH,), jnp.float32),
            pltpu.VMEM_SHARED((N, D), jnp.bfloat16),
            pltpu.VMEM_SHARED((N,), jnp.float32),
            [pltpu.SemaphoreType.DMA] * NBUF,
        ],
    )
    def k(x_hbm, row_hbm, col_hbm, agg_hbm, deg_hbm, rowi_v, coli_v,
          bufs, zd_v, ones_v, acc_sp, deg_sp, gsem):
        c = lax.axis_index("c")
        s = lax.axis_index("s")
        wid = s * NC + c

        zero16 = jnp.zeros((16,), jnp.float32)
        zero32 = jnp.zeros((32,), jnp.bfloat16)

        def zrow(r, _):
            def zcol(q, _):
                bufs[0][r, pl.ds(q * 32, 32)] = zero32
                return _
            return lax.fori_loop(0, D // 32, zcol, _)
        lax.fori_loop(0, CH, zrow, None)

        def zdeg(r, _):
            zd_v[pl.ds(r * 16, 16)] = zero16
            return _
        lax.fori_loop(0, DGB // 16, zdeg, None)

        def fones(r, _):
            ones_v[pl.ds(r * 16, 16)] = jnp.ones((16,), jnp.float32)
            return _
        lax.fori_loop(0, CH // 16, fones, None)

        def zacc(b, _):
            pltpu.sync_copy(bufs[0],
                            acc_sp.at[pl.ds(s * RPT + b * CH, CH)])
            return _
        lax.fori_loop(0, RPT // CH, zacc, None)
        pltpu.sync_copy(bufs[0].at[pl.ds(0, RPT % CH)],
                        acc_sp.at[pl.ds(s * RPT + RPT - RPT % CH,
                                        RPT % CH)])

        @pl.when(s < N // DGB)
        def _():
            pltpu.sync_copy(zd_v, deg_sp.at[pl.ds(s * DGB, DGB)])

        pltpu.sync_copy(row_hbm.at[wid], rowi_v)
        pltpu.sync_copy(col_hbm.at[wid], coli_v)

        plsc.subcore_barrier()

        def gather(i, j):
            pltpu.async_copy(x_hbm.at[rowi_v.at[i]], bufs[j], gsem[j])

        def wait_gather(i, j):
            pltpu.make_async_copy(x_hbm.at[rowi_v.at[i]], bufs[j],
                                  gsem[j]).wait()

        def scat(i, j):
            pltpu.sync_copy(bufs[j], acc_sp.at[coli_v.at[i]], add=True)
            pltpu.sync_copy(ones_v, deg_sp.at[coli_v.at[i]], add=True)

        for j in range(NBUF):
            gather(j, j)

        def group(gi, _):
            base = gi * NBUF
            for j in range(NBUF):
                wait_gather(base + j, j)
                scat(base + j, j)
                nxt = base + NBUF + j

                @pl.when(nxt < NCH)
                def _(j=j, nxt=nxt):
                    gather(nxt, j)
            return _
        lax.fori_loop(0, NG, group, None)

        plsc.subcore_barrier()

        pltpu.sync_copy(acc_sp.at[pl.ds(s * RPT, RPT)],
                        agg_hbm.at[c, pl.ds(s * RPT, RPT)])

        @pl.when(s < N // DGB)
        def _():
            pltpu.sync_copy(deg_sp.at[pl.ds(s * DGB, DGB)],
                            deg_hbm.at[c, pl.ds(s * DGB, DGB)])

    return k(x, row_idx, col_idx)


def _tc_combine_kernel(aggp_ref, degp_ref, x_ref, wn_ref, wr_ref, o_ref):
    inv_sqrt_d = np.float32(1.0 / np.sqrt(D))
    join_scale = np.float32(0.5 / np.sqrt(0.5))

    def norm_w(w):
        nrm = jnp.sqrt(jnp.sum(w * w, axis=1, keepdims=True))
        sc = np.float32(1e-4) + nrm * inv_sqrt_d
        return (w / sc) * inv_sqrt_d

    feat = (aggp_ref[0].astype(jnp.float32)
            + aggp_ref[1].astype(jnp.float32))
    deg = degp_ref[0] + degp_ref[1]
    dinv = jnp.where(deg > 0, lax.rsqrt(jnp.maximum(deg, 1.0)), 0.0)
    agg = feat * dinv

    wn = norm_w(wn_ref[...])
    wr = norm_w(wr_ref[...])
    dn = (((1,), (1,)), ((), ()))
    neigh = lax.dot_general(agg, wn, dn, preferred_element_type=jnp.float32)
    res = lax.dot_general(x_ref[...], wr, dn,
                          preferred_element_type=jnp.float32)
    o_ref[...] = jnp.where(deg > 0, (res + neigh) * join_scale, res)


def _tc_combine(aggp, degp, x, w_neigh, w_res):
    blk = 1000
    grid = N // blk
    return pl.pallas_call(
        _tc_combine_kernel,
        grid=(grid,),
        in_specs=[
            pl.BlockSpec((NC, blk, D), lambda i: (0, i, 0)),
            pl.BlockSpec((NC, blk, 1), lambda i: (0, i, 0)),
            pl.BlockSpec((blk, D), lambda i: (i, 0)),
            pl.BlockSpec((D, D), lambda i: (0, 0)),
            pl.BlockSpec((D, D), lambda i: (0, 0)),
        ],
        out_specs=pl.BlockSpec((blk, D), lambda i: (i, 0)),
        out_shape=jax.ShapeDtypeStruct((N, D), jnp.float32),
    )(aggp, degp, x, w_neigh, w_res)


@jax.jit
def kernel(x, edge_index, W_neigh, W_res):
    x = x.astype(jnp.float32)
    ei = edge_index.astype(jnp.int32)
    row = ei[0].reshape(NW, NCH, CH)
    col = ei[1].reshape(NW, NCH, CH)
    aggp, degp = _sc_aggregate(x.astype(jnp.bfloat16), row, col)
    return _tc_combine(aggp, degp.reshape(NC, N, 1), x, W_neigh, W_res)


# speedup vs baseline: 1.4980x; 1.0207x over previous
import functools

import jax
import jax.numpy as jnp
import numpy as np
from jax import lax
from jax.experimental import pallas as pl
from jax.experimental.pallas import tpu as pltpu
from jax.experimental.pallas import tpu_sc as plsc

N = 10000
E = 320000
D = 128
NC = 2
NS = 16
NW = NC * NS
EPT = E // NW
CH = 80
NCH = EPT // CH
NBUF = 5
NG = NCH // NBUF
RPT = N // NS
DGB = 1000


def _sc_aggregate(x, row_idx, col_idx):
    mesh = plsc.VectorSubcoreMesh(
        core_axis_name="c", subcore_axis_name="s", num_cores=NC,
        num_subcores=NS)

    @functools.partial(
        pl.kernel,
        out_type=(
            jax.ShapeDtypeStruct((NC, N, D), jnp.bfloat16),
            jax.ShapeDtypeStruct((NC, N), jnp.float32),
        ),
        mesh=mesh,
        compiler_params=pltpu.CompilerParams(use_tc_tiling_on_sc=False),
        scratch_types=[
            pltpu.VMEM((NCH, CH), jnp.int32),
            pltpu.VMEM((NCH, CH), jnp.int32),
            [pltpu.VMEM((CH, D), jnp.bfloat16)] * NBUF,
            pltpu.VMEM((DGB,), jnp.float32),
            pltpu.VMEM((CH,), jnp.float32),
            pltpu.VMEM_SHARED((N, D), jnp.bfloat16),
            pltpu.VMEM_SHARED((N,), jnp.float32),
            [pltpu.SemaphoreType.DMA] * NBUF,
            pltpu.SemaphoreType.DMA,
            pltpu.SemaphoreType.DMA,
        ],
    )
    def k(x_hbm, row_hbm, col_hbm, agg_hbm, deg_hbm, rowi_v, coli_v,
          bufs, zd_v, ones_v, acc_sp, deg_sp, gsem, fsem, dsem):
        c = lax.axis_index("c")
        s = lax.axis_index("s")
        wid = s * NC + c

        zero16 = jnp.zeros((16,), jnp.float32)
        zero32 = jnp.zeros((32,), jnp.bfloat16)

        def zrow(r, _):
            def zcol(q, _):
                bufs[0][r, pl.ds(q * 32, 32)] = zero32
                return _
            return lax.fori_loop(0, D // 32, zcol, _)
        lax.fori_loop(0, CH, zrow, None)

        def zdeg(r, _):
            zd_v[pl.ds(r * 16, 16)] = zero16
            return _
        lax.fori_loop(0, DGB // 16, zdeg, None)

        def fones(r, _):
            ones_v[pl.ds(r * 16, 16)] = jnp.ones((16,), jnp.float32)
            return _
        lax.fori_loop(0, CH // 16, fones, None)

        def zacc(b, _):
            pltpu.sync_copy(bufs[0],
                            acc_sp.at[pl.ds(s * RPT + b * CH, CH)])
            return _
        lax.fori_loop(0, RPT // CH, zacc, None)
        pltpu.sync_copy(bufs[0].at[pl.ds(0, RPT % CH)],
                        acc_sp.at[pl.ds(s * RPT + RPT - RPT % CH,
                                        RPT % CH)])

        @pl.when(s < N // DGB)
        def _():
            pltpu.sync_copy(zd_v, deg_sp.at[pl.ds(s * DGB, DGB)])

        pltpu.sync_copy(row_hbm.at[wid], rowi_v)
        pltpu.sync_copy(col_hbm.at[wid], coli_v)

        plsc.subcore_barrier()

        def gather(i, j):
            pltpu.async_copy(x_hbm.at[rowi_v.at[i]], bufs[j], gsem[j])

        def wait_gather(i, j):
            pltpu.make_async_copy(x_hbm.at[rowi_v.at[i]], bufs[j],
                                  gsem[j]).wait()

        def scat(i, j):
            f = pltpu.async_copy(bufs[j], acc_sp.at[coli_v.at[i]], fsem,
                                 add=True)
            d = pltpu.async_copy(ones_v, deg_sp.at[coli_v.at[i]], dsem,
                                 add=True)
            f.wait()
            d.wait()

        for j in range(NBUF):
            gather(j, j)

        def group(gi, _):
            base = gi * NBUF
            for j in range(NBUF):
                wait_gather(base + j, j)
                scat(base + j, j)
                nxt = base + NBUF + j

                @pl.when(nxt < NCH)
                def _(j=j, nxt=nxt):
                    gather(nxt, j)
            return _
        lax.fori_loop(0, NG, group, None)

        plsc.subcore_barrier()

        pltpu.sync_copy(acc_sp.at[pl.ds(s * RPT, RPT)],
                        agg_hbm.at[c, pl.ds(s * RPT, RPT)])

        @pl.when(s < N // DGB)
        def _():
            pltpu.sync_copy(deg_sp.at[pl.ds(s * DGB, DGB)],
                            deg_hbm.at[c, pl.ds(s * DGB, DGB)])

    return k(x, row_idx, col_idx)


def _tc_combine_kernel(aggp_ref, degp_ref, x_ref, wn_ref, wr_ref, o_ref):
    inv_sqrt_d = np.float32(1.0 / np.sqrt(D))
    join_scale = np.float32(0.5 / np.sqrt(0.5))

    def norm_w(w):
        nrm = jnp.sqrt(jnp.sum(w * w, axis=1, keepdims=True))
        sc = np.float32(1e-4) + nrm * inv_sqrt_d
        return (w / sc) * inv_sqrt_d

    feat = (aggp_ref[0].astype(jnp.float32)
            + aggp_ref[1].astype(jnp.float32))
    deg = degp_ref[0] + degp_ref[1]
    dinv = jnp.where(deg > 0, lax.rsqrt(jnp.maximum(deg, 1.0)), 0.0)
    agg = feat * dinv

    wn = norm_w(wn_ref[...])
    wr = norm_w(wr_ref[...])
    dn = (((1,), (1,)), ((), ()))
    neigh = lax.dot_general(agg, wn, dn, preferred_element_type=jnp.float32)
    res = lax.dot_general(x_ref[...], wr, dn,
                          preferred_element_type=jnp.float32)
    o_ref[...] = jnp.where(deg > 0, (res + neigh) * join_scale, res)


def _tc_combine(aggp, degp, x, w_neigh, w_res):
    blk = 1000
    grid = N // blk
    return pl.pallas_call(
        _tc_combine_kernel,
        grid=(grid,),
        in_specs=[
            pl.BlockSpec((NC, blk, D), lambda i: (0, i, 0)),
            pl.BlockSpec((NC, blk, 1), lambda i: (0, i, 0)),
            pl.BlockSpec((blk, D), lambda i: (i, 0)),
            pl.BlockSpec((D, D), lambda i: (0, 0)),
            pl.BlockSpec((D, D), lambda i: (0, 0)),
        ],
        out_specs=pl.BlockSpec((blk, D), lambda i: (i, 0)),
        out_shape=jax.ShapeDtypeStruct((N, D), jnp.float32),
    )(aggp, degp, x, w_neigh, w_res)


@jax.jit
def kernel(x, edge_index, W_neigh, W_res):
    x = x.astype(jnp.float32)
    ei = edge_index.astype(jnp.int32)
    row = ei[0].reshape(NW, NCH, CH)
    col = ei[1].reshape(NW, NCH, CH)
    aggp, degp = _sc_aggregate(x.astype(jnp.bfloat16), row, col)
    return _tc_combine(aggp, degp.reshape(NC, N, 1), x, W_neigh, W_res)
